# baseline (speedup 1.0000x reference)
import functools

import jax
import jax.numpy as jnp
from jax import lax
from jax.experimental import pallas as pl
from jax.experimental.pallas import tpu as pltpu
from jax.experimental.pallas import tpu_sc as plsc

E = 64
N_TOKENS = 262144
NUM_TILES = 16
CHUNK = N_TOKENS // NUM_TILES
LANES = 16
N_EVEC = E // LANES
GROUP = NUM_TILES * LANES


def _body(idx_hbm, group_hbm, out_hbm, chunk, partial, shared, accv, gv, outv,
          sem, psem):
    s = lax.axis_index("s")

    h = pltpu.async_copy(idx_hbm.at[pl.ds(s * CHUNK, CHUNK)], chunk, sem)

    @pl.when(s < N_EVEC)
    def _():
        pltpu.sync_copy(group_hbm, gv)

    h.wait()

    lane = lax.iota(jnp.int32, LANES)
    e_vecs = [lane + jnp.int32(j * LANES) for j in range(N_EVEC)]
    zero = jnp.zeros((LANES,), jnp.int32)

    def _step(_, carry):
        step, ps = carry[0], list(carry[1:])
        off = step - jnp.int32(1)
        for j in range(N_EVEC):
            val = plsc.load_gather(chunk, [ps[j] + off])
            ps[j] = jnp.where(val <= e_vecs[j], ps[j] + step, ps[j])
        return (jnp.maximum(step // jnp.int32(2), jnp.int32(1)), *ps)

    carry = lax.fori_loop(
        0, CHUNK.bit_length(), _step,
        (jnp.int32(CHUNK // 2), zero, zero, zero, zero),
    )
    pubs = []
    for j in range(N_EVEC):
        pos = carry[1 + j]
        partial[pl.ds(j * LANES, LANES)] = pos
        pubs.append(
            pltpu.async_copy(
                partial.at[pl.ds(j * LANES, LANES)],
                shared.at[pl.ds(j * GROUP + s * LANES, LANES)],
                psem,
            )
        )
    for p in pubs:
        p.wait()
    plsc.subcore_barrier()

    @pl.when(s < N_EVEC)
    def _():
        pltpu.sync_copy(shared.at[pl.ds(s * GROUP, GROUP)], accv)
        acc0 = plsc.load_gather(gv, [lane + s * LANES])

        def _acc(r, a):
            return a + plsc.load_gather(accv, [lane + r * LANES])

        outv[...] = lax.fori_loop(0, NUM_TILES, _acc, acc0)
        pltpu.sync_copy(outv, out_hbm.at[pl.ds(s * LANES, LANES)])


@jax.jit
def _collect(indices_expert, expert_group_list):
    mesh = plsc.VectorSubcoreMesh(
        core_axis_name="c", subcore_axis_name="s", num_cores=1
    )
    k = functools.partial(
        pl.kernel,
        mesh=mesh,
        out_type=jax.ShapeDtypeStruct((E,), jnp.int32),
        scratch_types=[
            pltpu.VMEM((CHUNK,), jnp.int32),
            pltpu.VMEM((E,), jnp.int32),
            pltpu.VMEM_SHARED((N_EVEC * GROUP,), jnp.int32),
            pltpu.VMEM((GROUP,), jnp.int32),
            pltpu.VMEM((E,), jnp.int32),
            pltpu.VMEM((LANES,), jnp.int32),
            pltpu.SemaphoreType.DMA,
            pltpu.SemaphoreType.DMA,
        ],
        compiler_params=pltpu.CompilerParams(needs_layout_passes=False),
    )(_body)
    return k(indices_expert, expert_group_list)


def kernel(indices_expert, expert_group_list):
    out = _collect(
        indices_expert.astype(jnp.int32), expert_group_list.astype(jnp.int32)
    )
    return out.astype(expert_group_list.dtype)


# speedup vs baseline: 1.0013x; 1.0013x over previous
import functools

import jax
import jax.numpy as jnp
from jax import lax
from jax.experimental import pallas as pl
from jax.experimental.pallas import tpu as pltpu
from jax.experimental.pallas import tpu_sc as plsc

E = 64
N_TOKENS = 262144
NUM_TILES = 16
CHUNK = N_TOKENS // NUM_TILES
LANES = 16
N_EVEC = E // LANES
GROUP = NUM_TILES * LANES


def _body(idx_hbm, group_hbm, out_hbm, chunk, partial, shared, accv, gv, outv,
          sem, psem):
    s = lax.axis_index("s")

    h = pltpu.async_copy(idx_hbm.at[pl.ds(s * CHUNK, CHUNK)], chunk, sem)

    @pl.when(s < N_EVEC)
    def _():
        pltpu.sync_copy(group_hbm, gv)

    h.wait()

    lane = lax.iota(jnp.int32, LANES)
    e_vecs = [lane + jnp.int32(j * LANES) for j in range(N_EVEC)]
    zero = jnp.zeros((LANES,), jnp.int32)

    def _step(_, carry):
        step, ps = carry[0], list(carry[1:])
        off = step - jnp.int32(1)
        for j in range(N_EVEC):
            val = plsc.load_gather(chunk, [ps[j] + off])
            ps[j] = jnp.where(val <= e_vecs[j], ps[j] + step, ps[j])
        return (jnp.maximum(step // jnp.int32(2), jnp.int32(1)), *ps)

    carry = lax.fori_loop(
        0, CHUNK.bit_length(), _step,
        (jnp.int32(CHUNK // 2), zero, zero, zero, zero),
    )
    for j in range(N_EVEC):
        partial[pl.ds(j * LANES, LANES)] = carry[1 + j]
    pltpu.async_copy(partial, shared.at[pl.ds(s * E, E)], psem).wait()
    plsc.subcore_barrier()

    @pl.when(s < N_EVEC)
    def _():
        pltpu.sync_copy(shared, accv)
        acc0 = plsc.load_gather(gv, [lane + s * LANES])

        def _acc(r, a):
            return a + plsc.load_gather(accv, [lane + (r * E + s * LANES)])

        outv[...] = lax.fori_loop(0, NUM_TILES, _acc, acc0)
        pltpu.sync_copy(outv, out_hbm.at[pl.ds(s * LANES, LANES)])


@jax.jit
def _collect(indices_expert, expert_group_list):
    mesh = plsc.VectorSubcoreMesh(
        core_axis_name="c", subcore_axis_name="s", num_cores=1
    )
    k = functools.partial(
        pl.kernel,
        mesh=mesh,
        out_type=jax.ShapeDtypeStruct((E,), jnp.int32),
        scratch_types=[
            pltpu.VMEM((CHUNK,), jnp.int32),
            pltpu.VMEM((E,), jnp.int32),
            pltpu.VMEM_SHARED((NUM_TILES * E,), jnp.int32),
            pltpu.VMEM((NUM_TILES * E,), jnp.int32),
            pltpu.VMEM((E,), jnp.int32),
            pltpu.VMEM((LANES,), jnp.int32),
            pltpu.SemaphoreType.DMA,
            pltpu.SemaphoreType.DMA,
        ],
        compiler_params=pltpu.CompilerParams(needs_layout_passes=False),
    )(_body)
    return k(indices_expert, expert_group_list)


def kernel(indices_expert, expert_group_list):
    out = _collect(
        indices_expert.astype(jnp.int32), expert_group_list.astype(jnp.int32)
    )
    return out.astype(expert_group_list.dtype)
